# baseline (speedup 1.0000x reference)
import functools

import jax
import jax.numpy as jnp
from jax import lax
from jax.experimental import pallas as pl
from jax.experimental.pallas import tpu as pltpu
from jax.experimental.pallas import tpu_sc as plsc

_BATCH = 16384
_FEAT = 64
_NC = 2
_NS = 16
_NW = _NC * _NS
_SPW = _BATCH // _NW
_NGRP = _SPW // 16
_LANES = 16
_IDBITS = 14
_IDMASK = (1 << _IDBITS) - 1
_BLK = 128
_IMAX = 2**31 - 1


def _make_kernel():
    mesh = plsc.VectorSubcoreMesh(core_axis_name="c", subcore_axis_name="s")

    @functools.partial(
        pl.kernel,
        mesh=mesh,
        out_type=jax.ShapeDtypeStruct((_NW, _LANES), jnp.float32),
        compiler_params=pltpu.CompilerParams(
            use_tc_tiling_on_sc=True, needs_layout_passes=False),
        scratch_types=[
            pltpu.VMEM((_SPW,), jnp.int32),
            pltpu.VMEM((_SPW * _FEAT,), jnp.float32),
            pltpu.VMEM((2, _FEAT, _BLK), jnp.float32),
            pltpu.VMEM((_LANES,), jnp.float32),
            pltpu.SemaphoreType.DMA,
            pltpu.SemaphoreType.DMA,
            pltpu.SemaphoreType.DMA,
        ],
    )
    def sc_center_loss(feats_hbm, keys_hbm, centersT_hbm, out_hbm,
                       key_v, feat_v, cwin_v, out_v, fsem, w0, w1):
        wsems = (w0, w1)
        wid = lax.axis_index("s") * _NC + lax.axis_index("c")
        base = wid * _SPW

        pltpu.sync_copy(keys_hbm.at[pl.ds(base, _SPW)], key_v)

        iota = lax.iota(jnp.int32, _LANES)

        def fire_feats(g, carry):
            key_vec = key_v[pl.ds(g * _LANES, _LANES)]
            off_vec = lax.shift_left(
                lax.bitwise_and(key_vec, _IDMASK), 6)
            for lane in range(_LANES):
                off = pl.multiple_of(off_vec[lane], _FEAT)
                pltpu.async_copy(
                    feats_hbm.at[pl.ds(off, _FEAT)],
                    feat_v.at[pl.ds((g * _LANES + lane) * _FEAT, _FEAT)],
                    fsem)
            return carry
        lax.fori_loop(0, _NGRP, fire_feats, 0)

        def label_at(pos):
            vec = key_v[pl.ds(pl.multiple_of(
                (pos // _LANES) * _LANES, _LANES), _LANES)]
            return lax.shift_right_logical(vec[pos % _LANES], _IDBITS)

        lo_blk = label_at(0) // _BLK
        hi_blk = label_at(_SPW - 1) // _BLK
        nwin = hi_blk - lo_blk + 1

        def fire_win(w, buf):
            off = pl.multiple_of((lo_blk + w) * _BLK, _BLK)
            pltpu.async_copy(
                centersT_hbm.at[:, pl.ds(off, _BLK)],
                cwin_v.at[buf], wsems[buf])

        def drain_win(buf):
            pltpu.make_async_copy(
                centersT_hbm.at[:, pl.ds(0, _BLK)],
                cwin_v.at[buf], wsems[buf]).wait()

        fire_win(0, 0)

        @pl.when(1 < nwin)
        def _():
            fire_win(1, 1)

        pltpu.make_async_copy(
            feats_hbm.at[pl.ds(0, _SPW * _FEAT)], feat_v, fsem).wait()

        def first_label(g):
            gc = lax.min(g, _NGRP - 1)
            vec = key_v[pl.ds(gc * _LANES, _LANES)]
            return lax.shift_right_logical(vec[0], _IDBITS)

        def last_label(g):
            gc = lax.max(lax.min(g, _NGRP - 1), 0)
            vec = key_v[pl.ds(gc * _LANES, _LANES)]
            return lax.shift_right_logical(vec[15], _IDBITS)

        def window_compute(w, buf, g0, acc0):
            win_lo = (lo_blk + w) * _BLK
            win_hi = win_lo + _BLK

            lo = g0
            hi = jnp.int32(_NGRP)
            for _ in range(6):
                mid = (lo + hi) // 2
                ok = jnp.logical_and(mid < _NGRP, first_label(mid) < win_hi)
                lo = jnp.where(ok, mid + 1, lo)
                hi = jnp.where(ok, hi, mid)
            g_end = lo

            def gbody(g, acc):
                key_vec = key_v[pl.ds(g * _LANES, _LANES)]
                lab_vec = lax.shift_right_logical(key_vec, _IDBITS)
                mask = jnp.logical_and(lab_vec >= win_lo, lab_vec < win_hi)
                mvec = mask.astype(jnp.float32)
                cols = jnp.clip(lab_vec - win_lo, 0, _BLK - 1)
                frows = lax.shift_left(g * _LANES + iota, 6)
                for d in range(_FEAT):
                    c = plsc.load_gather(cwin_v.at[buf, d], [cols])
                    f = plsc.load_gather(feat_v, [frows + d])
                    dd = (f - c) * mvec
                    acc = acc + dd * dd
                return acc

            accf = lax.fori_loop(g0, g_end, gbody, acc0)
            straddle = jnp.logical_and(
                g_end > g0, last_label(g_end - 1) >= win_hi)
            gf = jnp.where(straddle, g_end - 1, g_end)
            return gf, accf

        def pair_body(p, carry):
            g, acc = carry
            w0_ = 2 * p
            drain_win(0)
            g, acc = window_compute(w0_, 0, g, acc)

            @pl.when(w0_ + 2 < nwin)
            def _():
                fire_win(w0_ + 2, 0)

            @pl.when(w0_ + 1 < nwin)
            def _():
                drain_win(1)
            g, acc = window_compute(w0_ + 1, 1, g, acc)

            @pl.when(w0_ + 3 < nwin)
            def _():
                fire_win(w0_ + 3, 1)
            return (g, acc)

        npairs = (nwin + 1) // 2
        zero = jnp.zeros((_LANES,), jnp.float32)
        _, acc = lax.fori_loop(0, npairs, pair_body, (jnp.int32(0), zero))

        out_v[...] = acc
        pltpu.sync_copy(out_v, out_hbm.at[wid])

    return sc_center_loss


_sc_center_loss = _make_kernel()


def kernel(feats, labels, centers):
    keys = lax.sort(
        lax.shift_left(labels.astype(jnp.int32), _IDBITS)
        | lax.iota(jnp.int32, _BATCH))
    partials = _sc_center_loss(
        feats.reshape(_BATCH * _FEAT), keys, centers.T)
    return jnp.sum(partials) * (1.0 / _BATCH)


# speedup vs baseline: 1.2707x; 1.2707x over previous
import functools

import jax
import jax.numpy as jnp
from jax import lax
from jax.experimental import pallas as pl
from jax.experimental.pallas import tpu as pltpu
from jax.experimental.pallas import tpu_sc as plsc

_BATCH = 16384
_FEAT = 64
_NC = 2
_NS = 16
_NW = _NC * _NS
_SPW = _BATCH // _NW
_NGRP = _SPW // 16
_LANES = 16
_IDBITS = 14
_IDMASK = (1 << _IDBITS) - 1
_BLK = 128
_WBLK = 3
_WIN = _WBLK * _BLK
_NBLK = 100096 // _BLK
_IMAX = 2**31 - 1


def _make_kernel():
    mesh = plsc.VectorSubcoreMesh(core_axis_name="c", subcore_axis_name="s")

    @functools.partial(
        pl.kernel,
        mesh=mesh,
        out_type=jax.ShapeDtypeStruct((_NW, _LANES), jnp.float32),
        compiler_params=pltpu.CompilerParams(
            use_tc_tiling_on_sc=True, needs_layout_passes=False),
        scratch_types=[
            pltpu.VMEM((_SPW,), jnp.int32),
            pltpu.VMEM((_SPW, _FEAT), jnp.float32),
            pltpu.VMEM((2, _FEAT, _WIN), jnp.float32),
            pltpu.VMEM((_LANES,), jnp.float32),
            pltpu.SemaphoreType.DMA,
            pltpu.SemaphoreType.DMA,
            pltpu.SemaphoreType.DMA,
        ],
    )
    def sc_center_loss(feats_hbm, keys_hbm, centersT_hbm, out_hbm,
                       key_v, feat_v, cwin_v, out_v, fsem, w0, w1):
        wsems = (w0, w1)
        wid = lax.axis_index("s") * _NC + lax.axis_index("c")
        base = wid * _SPW

        pltpu.sync_copy(keys_hbm.at[pl.ds(base, _SPW)], key_v)

        iota = lax.iota(jnp.int32, _LANES)

        def fire_feats(g, carry):
            key_vec = key_v[pl.ds(g * _LANES, _LANES)]
            lid_vec = lax.bitwise_and(key_vec, _IDMASK)
            for lane in range(_LANES):
                lid = lid_vec[lane]
                pltpu.async_copy(
                    feats_hbm.at[pl.ds(lid, 1)],
                    feat_v.at[pl.ds(g * _LANES + lane, 1)], fsem)
            return carry
        lax.fori_loop(0, _NGRP, fire_feats, 0)

        def label_at(pos):
            vec = key_v[pl.ds(pl.multiple_of(
                (pos // _LANES) * _LANES, _LANES), _LANES)]
            return lax.shift_right_logical(vec[pos % _LANES], _IDBITS)

        lo_blk = label_at(0) // _BLK
        hi_blk = label_at(_SPW - 1) // _BLK
        nwin = (hi_blk - lo_blk) // _WBLK + 1

        def win_base(w):
            return lax.min(lo_blk + w * _WBLK, _NBLK - _WBLK) * _BLK

        def fire_win(w, buf):
            off = pl.multiple_of(win_base(w), _BLK)
            pltpu.async_copy(
                centersT_hbm.at[:, pl.ds(off, _WIN)],
                cwin_v.at[buf], wsems[buf])

        def drain_win(buf):
            pltpu.make_async_copy(
                centersT_hbm.at[:, pl.ds(0, _WIN)],
                cwin_v.at[buf], wsems[buf]).wait()

        fire_win(0, 0)

        @pl.when(1 < nwin)
        def _():
            fire_win(1, 1)

        pltpu.make_async_copy(
            feats_hbm.at[pl.ds(0, _SPW)], feat_v, fsem).wait()

        def first_label(g):
            gc = lax.min(g, _NGRP - 1)
            vec = key_v[pl.ds(gc * _LANES, _LANES)]
            return lax.shift_right_logical(vec[0], _IDBITS)

        def last_label(g):
            gc = lax.max(lax.min(g, _NGRP - 1), 0)
            vec = key_v[pl.ds(gc * _LANES, _LANES)]
            return lax.shift_right_logical(vec[15], _IDBITS)

        def window_compute(w, buf, g0, acc0):
            win_lo = (lo_blk + w * _WBLK) * _BLK
            win_hi = win_lo + _WIN
            cbase = win_base(w)

            lo = g0
            hi = jnp.int32(_NGRP)
            for _ in range(6):
                mid = (lo + hi) // 2
                ok = jnp.logical_and(mid < _NGRP, first_label(mid) < win_hi)
                lo = jnp.where(ok, mid + 1, lo)
                hi = jnp.where(ok, hi, mid)
            g_end = lo

            def gbody(g, acc):
                key_vec = key_v[pl.ds(g * _LANES, _LANES)]
                lab_vec = lax.shift_right_logical(key_vec, _IDBITS)
                mask = jnp.logical_and(lab_vec >= win_lo, lab_vec < win_hi)
                mvec = mask.astype(jnp.float32)
                cols = jnp.clip(lab_vec - cbase, 0, _WIN - 1)
                rows = g * _LANES + iota
                for d in range(_FEAT):
                    dvec = jnp.full((_LANES,), d, jnp.int32)
                    c = plsc.load_gather(cwin_v.at[buf], [dvec, cols])
                    f = plsc.load_gather(feat_v, [rows, dvec])
                    dd = (f - c) * mvec
                    acc = acc + dd * dd
                return acc

            accf = lax.fori_loop(g0, g_end, gbody, acc0)
            straddle = jnp.logical_and(
                g_end > g0, last_label(g_end - 1) >= win_hi)
            gf = jnp.where(straddle, g_end - 1, g_end)
            return gf, accf

        def pair_body(p, carry):
            g, acc = carry
            w0_ = 2 * p
            drain_win(0)
            g, acc = window_compute(w0_, 0, g, acc)

            @pl.when(w0_ + 2 < nwin)
            def _():
                fire_win(w0_ + 2, 0)

            @pl.when(w0_ + 1 < nwin)
            def _():
                drain_win(1)
            g, acc = window_compute(w0_ + 1, 1, g, acc)

            @pl.when(w0_ + 3 < nwin)
            def _():
                fire_win(w0_ + 3, 1)
            return (g, acc)

        npairs = (nwin + 1) // 2
        zero = jnp.zeros((_LANES,), jnp.float32)
        _, acc = lax.fori_loop(0, npairs, pair_body, (jnp.int32(0), zero))

        out_v[...] = acc
        pltpu.sync_copy(out_v, out_hbm.at[wid])

    return sc_center_loss


_sc_center_loss = _make_kernel()


def kernel(feats, labels, centers):
    keys = lax.sort(
        lax.shift_left(labels.astype(jnp.int32), _IDBITS)
        | lax.iota(jnp.int32, _BATCH))
    partials = _sc_center_loss(feats, keys, centers.T)
    return jnp.sum(partials) * (1.0 / _BATCH)
